# baseline (speedup 1.0000x reference)
import jax
import jax.numpy as jnp
from jax.experimental import pallas as pl
from jax.experimental.pallas import tpu as pltpu

_BN_EPS = 1e-5
_B = 2
_BT = 8


def _conv_dot(zbuf, w_ref, b_ref, H, W, C):
    taps = [zbuf[kh:kh + H, kw:kw + W, :]
            for kh in range(3) for kw in range(3)]
    patches = jnp.concatenate(taps, axis=-1).reshape(H * W, 9 * C)
    acc = jnp.dot(patches, w_ref[...], preferred_element_type=jnp.float32)
    return acc + b_ref[...]


def _stats(p, HW):
    s = jnp.sum(p, axis=0, keepdims=True)
    sq = jnp.sum(p * p, axis=0, keepdims=True)
    return s, sq - s * s * (1.0 / HW)


def _make_conv0(H, W, C):
    HW = H * W

    def body(x_ref, w_ref, b_ref, y_ref, s_ref, m2_ref, zbuf):
        zbuf[...] = jnp.zeros(zbuf.shape, zbuf.dtype)
        for b in range(_B):
            xb = x_ref[b].reshape(C, H * W)
            t = jnp.transpose(xb, (1, 0))
            zbuf[1:H + 1, 1:W + 1, :] = t.reshape(H, W, C).astype(jnp.bfloat16)
            p = _conv_dot(zbuf, w_ref, b_ref, H, W, C)
            s, m2 = _stats(p, HW)
            y_ref[b] = p.astype(jnp.bfloat16)
            s_ref[b, 0] = s[0]
            m2_ref[b, 0] = m2[0]
    return body


def _pool2(p, op, H, W, C):
    a = p.reshape(H // 2, 2 * W, C)
    u = op(a[:, :W, :], a[:, W:, :])
    v = u.reshape(H // 2, W // 2, 2, C)
    return op(v[:, :, 0], v[:, :, 1])


def _make_conv1_pool(H, W, C):
    HW = H * W
    H2, W2 = H // 2, W // 2

    def body(y0_ref, sc_ref, sh_ref, w_ref, b_ref,
             mx_ref, mn_ref, s_ref, m2_ref, zbuf):
        scale = sc_ref[...].astype(jnp.bfloat16)
        shift = sh_ref[...].astype(jnp.bfloat16)
        zbuf[...] = jnp.zeros(zbuf.shape, zbuf.dtype)
        for b in range(_B):
            z = y0_ref[b] * scale + shift
            zbuf[1:H + 1, 1:W + 1, :] = z.reshape(H, W, C)
            p = _conv_dot(zbuf, w_ref, b_ref, H, W, C)
            s, m2 = _stats(p, HW)
            s_ref[b, 0] = s[0]
            m2_ref[b, 0] = m2[0]
            pb = p.astype(jnp.bfloat16)
            mx = _pool2(pb, jnp.maximum, H, W, C)
            mn = _pool2(pb, jnp.minimum, H, W, C)
            mx_ref[b] = mx.reshape(H2 * W2, C)
            mn_ref[b] = mn.reshape(H2 * W2, C)
    return body


def _make_tail(HW4, C):
    def body(mx_ref, mn_ref, sc_ref, sh_ref, o_ref):
        sc = sc_ref[...]
        sh = sh_ref[...]
        mxv = mx_ref[...].astype(jnp.float32) * sc
        mnv = mn_ref[...].astype(jnp.float32) * sc
        o = jnp.where(sc > 0, mxv, mnv) + sh
        o_ref[...] = jnp.transpose(o, (0, 2, 1))
    return body


def _bn_combine(s, m2, gamma, beta, HW):
    N = s.shape[0]
    m_n = s / HW
    mean = jnp.mean(m_n, axis=0)
    var = (jnp.sum(m2, axis=0)
           + HW * jnp.sum((m_n - mean) ** 2, axis=0)) / (N * HW)
    scale = gamma * jax.lax.rsqrt(var + _BN_EPS)
    shift = beta - mean * scale
    return scale, shift


def kernel(x, w2d_0, b_0, gamma_0, beta_0, w2d_1, b_1, gamma_1, beta_1):
    N, C, H, W = x.shape
    HW = H * W
    H2, W2 = H // 2, W // 2
    HW4 = H2 * W2
    KKC = 9 * C
    w0 = w2d_0.astype(jnp.bfloat16)
    w1 = w2d_1.astype(jnp.bfloat16)

    y0, s0, m20 = pl.pallas_call(
        _make_conv0(H, W, C),
        out_shape=(jax.ShapeDtypeStruct((N, HW, C), jnp.bfloat16),
                   jax.ShapeDtypeStruct((N, 1, C), jnp.float32),
                   jax.ShapeDtypeStruct((N, 1, C), jnp.float32)),
        grid=(N // _B,),
        in_specs=[pl.BlockSpec((_B, C, H, W), lambda n: (n, 0, 0, 0)),
                  pl.BlockSpec((KKC, C), lambda n: (0, 0)),
                  pl.BlockSpec((1, C), lambda n: (0, 0))],
        out_specs=(pl.BlockSpec((_B, HW, C), lambda n: (n, 0, 0)),
                   pl.BlockSpec((_B, 1, C), lambda n: (n, 0, 0)),
                   pl.BlockSpec((_B, 1, C), lambda n: (n, 0, 0))),
        scratch_shapes=[pltpu.VMEM((H + 2, W + 2, C), jnp.bfloat16)],
        compiler_params=pltpu.CompilerParams(
            dimension_semantics=("parallel",)),
    )(x, w0, b_0.reshape(1, C))
    scale0, shift0 = _bn_combine(s0[:, 0, :], m20[:, 0, :], gamma_0, beta_0, HW)

    mx, mn, s1, m21 = pl.pallas_call(
        _make_conv1_pool(H, W, C),
        out_shape=(jax.ShapeDtypeStruct((N, HW4, C), jnp.bfloat16),
                   jax.ShapeDtypeStruct((N, HW4, C), jnp.bfloat16),
                   jax.ShapeDtypeStruct((N, 1, C), jnp.float32),
                   jax.ShapeDtypeStruct((N, 1, C), jnp.float32)),
        grid=(N // _B,),
        in_specs=[pl.BlockSpec((_B, HW, C), lambda n: (n, 0, 0)),
                  pl.BlockSpec((1, C), lambda n: (0, 0)),
                  pl.BlockSpec((1, C), lambda n: (0, 0)),
                  pl.BlockSpec((KKC, C), lambda n: (0, 0)),
                  pl.BlockSpec((1, C), lambda n: (0, 0))],
        out_specs=(pl.BlockSpec((_B, HW4, C), lambda n: (n, 0, 0)),
                   pl.BlockSpec((_B, HW4, C), lambda n: (n, 0, 0)),
                   pl.BlockSpec((_B, 1, C), lambda n: (n, 0, 0)),
                   pl.BlockSpec((_B, 1, C), lambda n: (n, 0, 0))),
        scratch_shapes=[pltpu.VMEM((H + 2, W + 2, C), jnp.bfloat16)],
        compiler_params=pltpu.CompilerParams(
            dimension_semantics=("parallel",)),
    )(y0, scale0.reshape(1, C), shift0.reshape(1, C), w1, b_1.reshape(1, C))
    scale1, shift1 = _bn_combine(s1[:, 0, :], m21[:, 0, :], gamma_1, beta_1, HW)

    out = pl.pallas_call(
        _make_tail(HW4, C),
        out_shape=jax.ShapeDtypeStruct((N, C, HW4), jnp.float32),
        grid=(N // _BT,),
        in_specs=[pl.BlockSpec((_BT, HW4, C), lambda n: (n, 0, 0)),
                  pl.BlockSpec((_BT, HW4, C), lambda n: (n, 0, 0)),
                  pl.BlockSpec((1, C), lambda n: (0, 0)),
                  pl.BlockSpec((1, C), lambda n: (0, 0))],
        out_specs=pl.BlockSpec((_BT, C, HW4), lambda n: (n, 0, 0)),
        compiler_params=pltpu.CompilerParams(
            dimension_semantics=("parallel",)),
    )(mx, mn, scale1.reshape(1, C), shift1.reshape(1, C))
    return out.reshape(N, C, H2, W2)


# speedup vs baseline: 1.5807x; 1.5807x over previous
import jax
import jax.numpy as jnp
from jax.experimental import pallas as pl
from jax.experimental.pallas import tpu as pltpu

_BN_EPS = 1e-5
_B = 2


def _conv_dot(zbuf, w_ref, b_ref, H, W, C):
    taps = [zbuf[kh:kh + H, kw:kw + W, :]
            for kh in range(3) for kw in range(3)]
    patches = jnp.concatenate(taps, axis=-1).reshape(H * W, 9 * C)
    acc = jnp.dot(patches, w_ref[...], preferred_element_type=jnp.float32)
    return acc + b_ref[...]


def _stats(p, HW):
    s = jnp.sum(p, axis=0, keepdims=True)
    sq = jnp.sum(p * p, axis=0, keepdims=True)
    return s, sq - s * s * (1.0 / HW)


def _pool2(p, op, H, W, C):
    a = p.reshape(H // 2, 2 * W, C)
    u = op(a[:, :W, :], a[:, W:, :])
    v = u.reshape(H // 2, W // 2, 2, C)
    return op(v[:, :, 0], v[:, :, 1])


def _make_conv0(H, W, C):
    HW = H * W

    def body(x_ref, w_ref, b_ref, y_ref, s_ref, m2_ref, zbuf):
        zbuf[...] = jnp.zeros(zbuf.shape, zbuf.dtype)
        for b in range(_B):
            zbuf[1:H + 1, 1:W + 1, :] = x_ref[b]
            p = _conv_dot(zbuf, w_ref, b_ref, H, W, C)
            s, m2 = _stats(p, HW)
            y_ref[b] = p.astype(jnp.bfloat16)
            s_ref[b, 0] = s[0]
            m2_ref[b, 0] = m2[0]
    return body


def _make_conv1_pool(H, W, C):
    HW = H * W
    H2, W2 = H // 2, W // 2

    def body(y0_ref, sc_ref, sh_ref, w_ref, b_ref,
             mx_ref, mn_ref, s_ref, m2_ref, zbuf):
        scale = sc_ref[...]
        shift = sh_ref[...]
        zbuf[...] = jnp.zeros(zbuf.shape, zbuf.dtype)
        for b in range(_B):
            z = y0_ref[b].astype(jnp.float32) * scale + shift
            zbuf[1:H + 1, 1:W + 1, :] = z.reshape(H, W, C)
            p = _conv_dot(zbuf, w_ref, b_ref, H, W, C)
            s, m2 = _stats(p, HW)
            s_ref[b, 0] = s[0]
            m2_ref[b, 0] = m2[0]
            pb = p.astype(jnp.bfloat16)
            mx = _pool2(pb, jnp.maximum, H, W, C)
            mn = _pool2(pb, jnp.minimum, H, W, C)
            mx_ref[b] = mx.reshape(H2 * W2, C)
            mn_ref[b] = mn.reshape(H2 * W2, C)
    return body


def _bn_combine(s, m2, gamma, beta, HW):
    N = s.shape[0]
    m_n = s / HW
    mean = jnp.mean(m_n, axis=0)
    var = (jnp.sum(m2, axis=0)
           + HW * jnp.sum((m_n - mean) ** 2, axis=0)) / (N * HW)
    scale = gamma * jax.lax.rsqrt(var + _BN_EPS)
    shift = beta - mean * scale
    return scale, shift


def kernel(x, w2d_0, b_0, gamma_0, beta_0, w2d_1, b_1, gamma_1, beta_1):
    N, C, H, W = x.shape
    HW = H * W
    H2, W2 = H // 2, W // 2
    HW4 = H2 * W2
    KKC = 9 * C
    x_nhwc = jnp.transpose(x, (0, 2, 3, 1))

    y0, s0, m20 = pl.pallas_call(
        _make_conv0(H, W, C),
        out_shape=(jax.ShapeDtypeStruct((N, HW, C), jnp.bfloat16),
                   jax.ShapeDtypeStruct((N, 1, C), jnp.float32),
                   jax.ShapeDtypeStruct((N, 1, C), jnp.float32)),
        grid=(N // _B,),
        in_specs=[pl.BlockSpec((_B, H, W, C), lambda n: (n, 0, 0, 0)),
                  pl.BlockSpec((KKC, C), lambda n: (0, 0)),
                  pl.BlockSpec((1, C), lambda n: (0, 0))],
        out_specs=(pl.BlockSpec((_B, HW, C), lambda n: (n, 0, 0)),
                   pl.BlockSpec((_B, 1, C), lambda n: (n, 0, 0)),
                   pl.BlockSpec((_B, 1, C), lambda n: (n, 0, 0))),
        scratch_shapes=[pltpu.VMEM((H + 2, W + 2, C), jnp.float32)],
        compiler_params=pltpu.CompilerParams(
            dimension_semantics=("parallel",)),
    )(x_nhwc, w2d_0, b_0.reshape(1, C))
    scale0, shift0 = _bn_combine(s0[:, 0, :], m20[:, 0, :], gamma_0, beta_0, HW)

    mx, mn, s1, m21 = pl.pallas_call(
        _make_conv1_pool(H, W, C),
        out_shape=(jax.ShapeDtypeStruct((N, HW4, C), jnp.bfloat16),
                   jax.ShapeDtypeStruct((N, HW4, C), jnp.bfloat16),
                   jax.ShapeDtypeStruct((N, 1, C), jnp.float32),
                   jax.ShapeDtypeStruct((N, 1, C), jnp.float32)),
        grid=(N // _B,),
        in_specs=[pl.BlockSpec((_B, HW, C), lambda n: (n, 0, 0)),
                  pl.BlockSpec((1, C), lambda n: (0, 0)),
                  pl.BlockSpec((1, C), lambda n: (0, 0)),
                  pl.BlockSpec((KKC, C), lambda n: (0, 0)),
                  pl.BlockSpec((1, C), lambda n: (0, 0))],
        out_specs=(pl.BlockSpec((_B, HW4, C), lambda n: (n, 0, 0)),
                   pl.BlockSpec((_B, HW4, C), lambda n: (n, 0, 0)),
                   pl.BlockSpec((_B, 1, C), lambda n: (n, 0, 0)),
                   pl.BlockSpec((_B, 1, C), lambda n: (n, 0, 0))),
        scratch_shapes=[pltpu.VMEM((H + 2, W + 2, C), jnp.float32)],
        compiler_params=pltpu.CompilerParams(
            dimension_semantics=("parallel",)),
    )(y0, scale0.reshape(1, C), shift0.reshape(1, C), w2d_1, b_1.reshape(1, C))
    scale1, shift1 = _bn_combine(s1[:, 0, :], m21[:, 0, :], gamma_1, beta_1, HW)

    sc = scale1.reshape(1, 1, C)
    out_nhwc = (jnp.where(sc > 0,
                          mx.astype(jnp.float32) * sc,
                          mn.astype(jnp.float32) * sc)
                + shift1.reshape(1, 1, C))
    return jnp.transpose(out_nhwc.reshape(N, H2, W2, C), (0, 3, 1, 2))
